# Initial kernel scaffold; baseline (speedup 1.0000x reference)
import jax
import jax.numpy as jnp
from jax.experimental import pallas as pl


def kernel(read_length, edge_index, W):
    raise NotImplementedError("write your pallas kernel here")


# speedup vs baseline: 54.5758x; 54.5758x over previous
import functools

import jax
import jax.numpy as jnp
from jax import lax
from jax.experimental import pallas as pl
from jax.experimental.pallas import tpu as pltpu
from jax.experimental.pallas import tpu_sc as plsc

N = 100000
E = 3200000
D = 128

NC = 2
NS = 16
NW = NC * NS

CH = 128
GR = 8
NR = E // CH
NG = NR // GR

NPAD = 100352
SL = NPAD // NS

assert E % (CH * GR) == 0 and NPAD % (NS * 8) == 0 and N <= NPAD


def _zero_slice(zbuf, spm, sid):
  def zb(i, _):
    zbuf[pl.ds(i * 16, 16)] = jnp.zeros((16,), jnp.float32)
    return 0
  lax.fori_loop(0, SL // 16, zb, 0)
  pltpu.sync_copy(zbuf, spm.at[pl.ds(sid * SL, SL)])


_MESH = plsc.VectorSubcoreMesh(
    core_axis_name="c", subcore_axis_name="s", num_cores=NC, num_subcores=NS)


@functools.partial(
    pl.kernel,
    out_type=(
        jax.ShapeDtypeStruct((NC * NPAD,), jnp.float32),
        jax.ShapeDtypeStruct((NC * NPAD,), jnp.float32),
    ),
    mesh=_MESH,
    scratch_types=dict(
        idx_s=pltpu.VMEM((GR, CH), jnp.int32),
        idx_d=pltpu.VMEM((GR, CH), jnp.int32),
        ones_v=pltpu.VMEM((CH,), jnp.float32),
        zbuf=pltpu.VMEM((SL,), jnp.float32),
        spm_out=pltpu.VMEM_SHARED((NPAD,), jnp.float32),
        spm_in=pltpu.VMEM_SHARED((NPAD,), jnp.float32),
    ),
)
def _hist_kernel(edges, degout_hbm, degin_hbm,
                 idx_s, idx_d, ones_v, zbuf, spm_out, spm_in):
  cid = lax.axis_index("c")
  sid = lax.axis_index("s")
  w = cid * NS + sid

  def ob(i, _):
    ones_v[pl.ds(i * 16, 16)] = jnp.ones((16,), jnp.float32)
    return 0
  lax.fori_loop(0, CH // 16, ob, 0)
  _zero_slice(zbuf, spm_out, sid)
  _zero_slice(zbuf, spm_in, sid)
  plsc.subcore_barrier()

  cnt = (NG - w + NW - 1) // NW

  def group(i, _):
    r = (w + i * NW) * GR
    pltpu.sync_copy(edges.at[pl.ds(r, GR), :], idx_s)
    pltpu.sync_copy(edges.at[pl.ds(NR + r, GR), :], idx_d)

    def sub(j, _):
      pltpu.sync_copy(ones_v, spm_out.at[idx_s.at[j]], add=True)
      pltpu.sync_copy(ones_v, spm_in.at[idx_d.at[j]], add=True)
      return 0
    lax.fori_loop(0, GR, sub, 0)
    return 0
  lax.fori_loop(0, cnt, group, 0)

  plsc.subcore_barrier()
  sl = pl.ds(sid * SL, SL)
  osl = pl.ds(cid * NPAD + sid * SL, SL)
  pltpu.sync_copy(spm_out.at[sl], degout_hbm.at[osl])
  pltpu.sync_copy(spm_in.at[sl], degin_hbm.at[osl])


@functools.partial(
    pl.kernel,
    out_type=jax.ShapeDtypeStruct((NC * NPAD,), jnp.float32),
    mesh=_MESH,
    scratch_types=dict(
        idx_s=pltpu.VMEM((GR, CH), jnp.int32),
        idx_d=pltpu.VMEM((GR, CH), jnp.int32),
        val_v=pltpu.VMEM((CH,), jnp.float32),
        zbuf=pltpu.VMEM((SL,), jnp.float32),
        spm_h=pltpu.VMEM_SHARED((NPAD,), jnp.float32),
        spm_agg=pltpu.VMEM_SHARED((NPAD,), jnp.float32),
    ),
)
def _agg_kernel(edges, h_hbm, agg_hbm,
                idx_s, idx_d, val_v, zbuf, spm_h, spm_agg):
  cid = lax.axis_index("c")
  sid = lax.axis_index("s")
  w = cid * NS + sid

  hsl = pl.ds(sid * SL, SL)
  pltpu.sync_copy(h_hbm.at[hsl], spm_h.at[hsl])
  _zero_slice(zbuf, spm_agg, sid)
  plsc.subcore_barrier()

  cnt = (NG - w + NW - 1) // NW

  def group(i, _):
    r = (w + i * NW) * GR
    pltpu.sync_copy(edges.at[pl.ds(r, GR), :], idx_s)
    pltpu.sync_copy(edges.at[pl.ds(NR + r, GR), :], idx_d)

    def sub(j, _):
      pltpu.sync_copy(spm_h.at[idx_s.at[j]], val_v)
      pltpu.sync_copy(val_v, spm_agg.at[idx_d.at[j]], add=True)
      return 0
    lax.fori_loop(0, GR, sub, 0)
    return 0
  lax.fori_loop(0, cnt, group, 0)

  plsc.subcore_barrier()
  pltpu.sync_copy(spm_agg.at[pl.ds(sid * SL, SL)],
                  agg_hbm.at[pl.ds(cid * NPAD + sid * SL, SL)])


def _h_body(d0_ref, d1_ref, rl_ref, h_ref):
  d = d0_ref[...] + d1_ref[...]
  h_ref[...] = (rl_ref[...] * (1.0 / 20000.0)) * lax.rsqrt(jnp.maximum(d, 1.0))


_R4 = 1024
_G4 = NPAD // _R4


def _out_body(a0_ref, a1_ref, di0_ref, di1_ref, w_ref, out_ref):
  a = a0_ref[...] + a1_ref[...]
  d = di0_ref[...] + di1_ref[...]
  av = a * lax.rsqrt(jnp.maximum(d, 1.0))
  w128 = jnp.broadcast_to(w_ref[...], (128, 128))
  rr = lax.broadcasted_iota(jnp.int32, (128, 128), 0)
  cc = lax.broadcasted_iota(jnp.int32, (128, 128), 1)
  eye = rr == cc
  for s in range(_R4 // 128):
    m = jnp.broadcast_to(av[s:s + 1, :], (128, 128))
    dg = jnp.where(eye, m, 0.0)
    blk = lax.dot_general(dg, w128, (((1,), (0,)), ((), ())),
                          preferred_element_type=jnp.float32)
    out_ref[pl.ds(s * 128, 128), :] = blk


def kernel(read_length, edge_index, W):
  edges = edge_index.reshape(2 * NR, CH)

  degout, degin = _hist_kernel(edges)
  degout = degout.reshape(NC, NPAD)
  degin = degin.reshape(NC, NPAD)

  rl = jnp.zeros((NPAD,), jnp.float32).at[:N].set(read_length)
  h = pl.pallas_call(
      _h_body,
      out_shape=jax.ShapeDtypeStruct((NPAD // 128, 128), jnp.float32),
  )(degout[0].reshape(NPAD // 128, 128),
    degout[1].reshape(NPAD // 128, 128),
    rl.reshape(NPAD // 128, 128))

  agg = _agg_kernel(edges, h.reshape(NPAD)).reshape(NC, NPAD)

  out = pl.pallas_call(
      _out_body,
      grid=(_G4,),
      in_specs=[
          pl.BlockSpec((_R4 // 128, 128), lambda g: (g, 0)),
          pl.BlockSpec((_R4 // 128, 128), lambda g: (g, 0)),
          pl.BlockSpec((_R4 // 128, 128), lambda g: (g, 0)),
          pl.BlockSpec((_R4 // 128, 128), lambda g: (g, 0)),
          pl.BlockSpec((1, 128), lambda g: (0, 0)),
      ],
      out_specs=pl.BlockSpec((_R4, 128), lambda g: (g, 0)),
      out_shape=jax.ShapeDtypeStruct((N, D), jnp.float32),
  )(agg[0].reshape(NPAD // 128, 128),
    agg[1].reshape(NPAD // 128, 128),
    degin[0].reshape(NPAD // 128, 128),
    degin[1].reshape(NPAD // 128, 128),
    W)
  return out


# speedup vs baseline: 78.3141x; 1.4350x over previous
import functools

import jax
import jax.numpy as jnp
from jax import lax
from jax.experimental import pallas as pl
from jax.experimental.pallas import tpu as pltpu
from jax.experimental.pallas import tpu_sc as plsc

N = 100000
E = 3200000
D = 128

NC = 2
NS = 16
NW = NC * NS

CH = 1024
NCHUNK = E // CH

NPAD = 100352
SL = NPAD // NS

assert E % CH == 0 and NPAD % (NS * 8) == 0 and N <= NPAD


def _zero_slice(zbuf, spm, sid):
  def zb(i, _):
    zbuf[pl.ds(i * 16, 16)] = jnp.zeros((16,), jnp.float32)
    return 0
  lax.fori_loop(0, SL // 16, zb, 0)
  pltpu.sync_copy(zbuf, spm.at[pl.ds(sid * SL, SL)])


def _rsqrt16(d):
  d = jnp.maximum(d, 1.0)
  i = plsc.bitcast(d, jnp.int32)
  y = plsc.bitcast(0x5F3759DF - (i >> 1), jnp.float32)
  for _ in range(3):
    y = y * (1.5 - 0.5 * d * y * y)
  return y


_MESH = plsc.VectorSubcoreMesh(
    core_axis_name="c", subcore_axis_name="s", num_cores=NC, num_subcores=NS)


@functools.partial(
    pl.kernel,
    out_type=(
        jax.ShapeDtypeStruct((NC * NPAD,), jnp.float32),
        jax.ShapeDtypeStruct((NC * NPAD,), jnp.float32),
    ),
    mesh=_MESH,
    compiler_params=pltpu.CompilerParams(needs_layout_passes=False),
    scratch_types=dict(
        idx_s=pltpu.VMEM((CH,), jnp.int32),
        idx_d=pltpu.VMEM((CH,), jnp.int32),
        ones_v=pltpu.VMEM((CH,), jnp.float32),
        zbuf=pltpu.VMEM((SL,), jnp.float32),
        spm_out=pltpu.VMEM_SHARED((NPAD,), jnp.float32),
        spm_in=pltpu.VMEM_SHARED((NPAD,), jnp.float32),
    ),
)
def _hist_kernel(edges, degout_hbm, degin_hbm,
                 idx_s, idx_d, ones_v, zbuf, spm_out, spm_in):
  cid = lax.axis_index("c")
  sid = lax.axis_index("s")
  w = cid * NS + sid

  def ob(i, _):
    ones_v[pl.ds(i * 16, 16)] = jnp.ones((16,), jnp.float32)
    return 0
  lax.fori_loop(0, CH // 16, ob, 0)
  _zero_slice(zbuf, spm_out, sid)
  _zero_slice(zbuf, spm_in, sid)
  plsc.subcore_barrier()

  cnt = (NCHUNK - w + NW - 1) // NW

  def chunk(i, _):
    base = (w + i * NW) * CH
    pltpu.sync_copy(edges.at[pl.ds(base, CH)], idx_s)
    pltpu.sync_copy(ones_v, spm_out.at[idx_s], add=True)
    pltpu.sync_copy(edges.at[pl.ds(E + base, CH)], idx_d)
    pltpu.sync_copy(ones_v, spm_in.at[idx_d], add=True)
    return 0
  lax.fori_loop(0, cnt, chunk, 0)

  plsc.subcore_barrier()
  sl = pl.ds(sid * SL, SL)
  osl = pl.ds(cid * NPAD + sid * SL, SL)
  pltpu.sync_copy(spm_out.at[sl], degout_hbm.at[osl])
  pltpu.sync_copy(spm_in.at[sl], degin_hbm.at[osl])


@functools.partial(
    pl.kernel,
    out_type=jax.ShapeDtypeStruct((NC * NPAD,), jnp.float32),
    mesh=_MESH,
    compiler_params=pltpu.CompilerParams(needs_layout_passes=False),
    scratch_types=dict(
        idx_s=pltpu.VMEM((CH,), jnp.int32),
        idx_d=pltpu.VMEM((CH,), jnp.int32),
        val_v=pltpu.VMEM((CH,), jnp.float32),
        d0_v=pltpu.VMEM((SL,), jnp.float32),
        d1_v=pltpu.VMEM((SL,), jnp.float32),
        h_v=pltpu.VMEM((SL,), jnp.float32),
        spm_h=pltpu.VMEM_SHARED((NPAD,), jnp.float32),
        spm_agg=pltpu.VMEM_SHARED((NPAD,), jnp.float32),
    ),
)
def _agg_kernel(edges, rl_hbm, degout_hbm, agg_hbm,
                idx_s, idx_d, val_v, d0_v, d1_v, h_v, spm_h, spm_agg):
  cid = lax.axis_index("c")
  sid = lax.axis_index("s")
  w = cid * NS + sid

  sl = pl.ds(sid * SL, SL)
  pltpu.sync_copy(degout_hbm.at[pl.ds(sid * SL, SL)], d0_v)
  pltpu.sync_copy(degout_hbm.at[pl.ds(NPAD + sid * SL, SL)], d1_v)
  pltpu.sync_copy(rl_hbm.at[sl], h_v)

  def hcomp(i, _):
    s16 = pl.ds(i * 16, 16)
    d = d0_v[s16] + d1_v[s16]
    h_v[s16] = h_v[s16] * (1.0 / 20000.0) * _rsqrt16(d)
    return 0
  lax.fori_loop(0, SL // 16, hcomp, 0)
  pltpu.sync_copy(h_v, spm_h.at[sl])

  def zb(i, _):
    d0_v[pl.ds(i * 16, 16)] = jnp.zeros((16,), jnp.float32)
    return 0
  lax.fori_loop(0, SL // 16, zb, 0)
  pltpu.sync_copy(d0_v, spm_agg.at[sl])
  plsc.subcore_barrier()

  cnt = (NCHUNK - w + NW - 1) // NW

  def chunk(i, _):
    base = (w + i * NW) * CH
    pltpu.sync_copy(edges.at[pl.ds(base, CH)], idx_s)
    pltpu.sync_copy(spm_h.at[idx_s], val_v)
    pltpu.sync_copy(edges.at[pl.ds(E + base, CH)], idx_d)
    pltpu.sync_copy(val_v, spm_agg.at[idx_d], add=True)
    return 0
  lax.fori_loop(0, cnt, chunk, 0)

  plsc.subcore_barrier()
  pltpu.sync_copy(spm_agg.at[sl], agg_hbm.at[pl.ds(cid * NPAD + sid * SL, SL)])


_R4 = 1024
_G4 = NPAD // _R4


def _out_body(a0_ref, a1_ref, di0_ref, di1_ref, w_ref, out_ref):
  a = a0_ref[...] + a1_ref[...]
  d = di0_ref[...] + di1_ref[...]
  av = a * lax.rsqrt(jnp.maximum(d, 1.0))
  w128 = jnp.broadcast_to(w_ref[...], (128, 128))
  rr = lax.broadcasted_iota(jnp.int32, (128, 128), 0)
  cc = lax.broadcasted_iota(jnp.int32, (128, 128), 1)
  eye = rr == cc
  for s in range(_R4 // 128):
    m = jnp.broadcast_to(av[s:s + 1, :], (128, 128))
    dg = jnp.where(eye, m, 0.0)
    blk = lax.dot_general(dg, w128, (((1,), (0,)), ((), ())),
                          preferred_element_type=jnp.float32)
    out_ref[pl.ds(s * 128, 128), :] = blk


def kernel(read_length, edge_index, W):
  edges = edge_index.reshape(2 * E)

  degout, degin = _hist_kernel(edges)
  degin = degin.reshape(NC, NPAD)

  rl = jnp.zeros((NPAD,), jnp.float32).at[:N].set(read_length)
  agg = _agg_kernel(edges, rl, degout).reshape(NC, NPAD)

  out = pl.pallas_call(
      _out_body,
      grid=(_G4,),
      in_specs=[
          pl.BlockSpec((_R4 // 128, 128), lambda g: (g, 0)),
          pl.BlockSpec((_R4 // 128, 128), lambda g: (g, 0)),
          pl.BlockSpec((_R4 // 128, 128), lambda g: (g, 0)),
          pl.BlockSpec((_R4 // 128, 128), lambda g: (g, 0)),
          pl.BlockSpec((1, 128), lambda g: (0, 0)),
      ],
      out_specs=pl.BlockSpec((_R4, 128), lambda g: (g, 0)),
      out_shape=jax.ShapeDtypeStruct((N, D), jnp.float32),
  )(agg[0].reshape(NPAD // 128, 128),
    agg[1].reshape(NPAD // 128, 128),
    degin[0].reshape(NPAD // 128, 128),
    degin[1].reshape(NPAD // 128, 128),
    W)
  return out


# speedup vs baseline: 122.0122x; 1.5580x over previous
import functools

import jax
import jax.numpy as jnp
from jax import lax
from jax.experimental import pallas as pl
from jax.experimental.pallas import tpu as pltpu
from jax.experimental.pallas import tpu_sc as plsc

N = 100000
E = 3200000
D = 128

NC = 2
NS = 16
NW = NC * NS

CH = 1024
CPW = 100
U = 4
NI = CPW // U
EP = NW * CPW * CH

NPAD = 100352
SL = NPAD // NS

assert CPW % U == 0 and NPAD % (NS * 8) == 0 and N + 64 <= NPAD and E <= EP


def _rsqrt16(d):
  d = jnp.maximum(d, 1.0)
  i = plsc.bitcast(d, jnp.int32)
  y = plsc.bitcast(0x5F3759DF - (i >> 1), jnp.float32)
  for _ in range(3):
    y = y * (1.5 - 0.5 * d * y * y)
  return y


_MESH = plsc.VectorSubcoreMesh(
    core_axis_name="c", subcore_axis_name="s", num_cores=NC, num_subcores=NS)

_SC_PARAMS = pltpu.CompilerParams(needs_layout_passes=False)


@functools.partial(
    pl.kernel,
    out_type=(
        jax.ShapeDtypeStruct((NC * NPAD,), jnp.float32),
        jax.ShapeDtypeStruct((NC * NPAD,), jnp.float32),
    ),
    mesh=_MESH,
    compiler_params=_SC_PARAMS,
    scratch_types=dict(
        idx_s=[pltpu.VMEM((CH,), jnp.int32) for _ in range(U)],
        idx_d=[pltpu.VMEM((CH,), jnp.int32) for _ in range(U)],
        ones_v=pltpu.VMEM((CH,), jnp.float32),
        zbuf=pltpu.VMEM((SL,), jnp.float32),
        spm_out=pltpu.VMEM_SHARED((NPAD,), jnp.float32),
        spm_in=pltpu.VMEM_SHARED((NPAD,), jnp.float32),
        sem_in=pltpu.SemaphoreType.DMA,
        sem_w=pltpu.SemaphoreType.DMA,
    ),
)
def _hist_kernel(edges, degout_hbm, degin_hbm,
                 idx_s, idx_d, ones_v, zbuf, spm_out, spm_in, sem_in, sem_w):
  cid = lax.axis_index("c")
  sid = lax.axis_index("s")
  w = cid * NS + sid

  def start_loads(j):
    for q in range(U):
      base = (w + (j * U + q) * NW) * CH
      pltpu.async_copy(edges.at[pl.ds(base, CH)], idx_s[q], sem_in)
      pltpu.async_copy(edges.at[pl.ds(EP + base, CH)], idx_d[q], sem_in)

  def wait_loads():
    for q in range(U):
      pltpu.make_async_copy(edges.at[pl.ds(0, CH)], idx_s[q], sem_in).wait()
      pltpu.make_async_copy(edges.at[pl.ds(0, CH)], idx_d[q], sem_in).wait()

  start_loads(0)

  def ob(i, _):
    ones_v[pl.ds(i * 16, 16)] = jnp.ones((16,), jnp.float32)
    return 0
  lax.fori_loop(0, CH // 16, ob, 0)

  def zb(i, _):
    zbuf[pl.ds(i * 16, 16)] = jnp.zeros((16,), jnp.float32)
    return 0
  lax.fori_loop(0, SL // 16, zb, 0)
  sl = pl.ds(sid * SL, SL)
  pltpu.sync_copy(zbuf, spm_out.at[sl])
  pltpu.sync_copy(zbuf, spm_in.at[sl])
  plsc.subcore_barrier()

  def chunk4(i, _):
    wait_loads()
    for q in range(U):
      pltpu.async_copy(ones_v, spm_out.at[idx_s[q]], sem_w, add=True)
      pltpu.async_copy(ones_v, spm_in.at[idx_d[q]], sem_w, add=True)
    for q in range(U):
      pltpu.make_async_copy(ones_v, spm_out.at[idx_s[q]], sem_w).wait()
      pltpu.make_async_copy(ones_v, spm_in.at[idx_d[q]], sem_w).wait()

    @pl.when(i + 1 < NI)
    def _():
      start_loads(i + 1)
    return 0
  lax.fori_loop(0, NI, chunk4, 0)

  plsc.subcore_barrier()
  osl = pl.ds(cid * NPAD + sid * SL, SL)
  pltpu.sync_copy(spm_out.at[sl], degout_hbm.at[osl])
  pltpu.sync_copy(spm_in.at[sl], degin_hbm.at[osl])


@functools.partial(
    pl.kernel,
    out_type=jax.ShapeDtypeStruct((NC * NPAD,), jnp.float32),
    mesh=_MESH,
    compiler_params=_SC_PARAMS,
    scratch_types=dict(
        idx_s=[pltpu.VMEM((CH,), jnp.int32) for _ in range(U)],
        idx_d=[pltpu.VMEM((CH,), jnp.int32) for _ in range(U)],
        val_v=[pltpu.VMEM((CH,), jnp.float32) for _ in range(U)],
        d0_v=pltpu.VMEM((SL,), jnp.float32),
        d1_v=pltpu.VMEM((SL,), jnp.float32),
        h_v=pltpu.VMEM((SL,), jnp.float32),
        spm_h=pltpu.VMEM_SHARED((NPAD,), jnp.float32),
        spm_agg=pltpu.VMEM_SHARED((NPAD,), jnp.float32),
        sem_in=pltpu.SemaphoreType.DMA,
        sem_g=pltpu.SemaphoreType.DMA,
        sem_w=pltpu.SemaphoreType.DMA,
    ),
)
def _agg_kernel(edges, rl_hbm, degout_hbm, agg_hbm,
                idx_s, idx_d, val_v, d0_v, d1_v, h_v,
                spm_h, spm_agg, sem_in, sem_g, sem_w):
  cid = lax.axis_index("c")
  sid = lax.axis_index("s")
  w = cid * NS + sid

  def start_loads(j):
    for q in range(U):
      base = (w + (j * U + q) * NW) * CH
      pltpu.async_copy(edges.at[pl.ds(base, CH)], idx_s[q], sem_in)
      pltpu.async_copy(edges.at[pl.ds(EP + base, CH)], idx_d[q], sem_in)

  start_loads(0)

  sl = pl.ds(sid * SL, SL)
  pltpu.sync_copy(degout_hbm.at[pl.ds(sid * SL, SL)], d0_v)
  pltpu.sync_copy(degout_hbm.at[pl.ds(NPAD + sid * SL, SL)], d1_v)
  pltpu.sync_copy(rl_hbm.at[sl], h_v)

  def hcomp(i, _):
    s16 = pl.ds(i * 16, 16)
    d = d0_v[s16] + d1_v[s16]
    h_v[s16] = h_v[s16] * (1.0 / 20000.0) * _rsqrt16(d)
    return 0
  lax.fori_loop(0, SL // 16, hcomp, 0)
  pltpu.sync_copy(h_v, spm_h.at[sl])

  def zb(i, _):
    d0_v[pl.ds(i * 16, 16)] = jnp.zeros((16,), jnp.float32)
    return 0
  lax.fori_loop(0, SL // 16, zb, 0)
  pltpu.sync_copy(d0_v, spm_agg.at[sl])
  plsc.subcore_barrier()

  def chunk4(i, _):
    for q in range(U):
      pltpu.make_async_copy(edges.at[pl.ds(0, CH)], idx_s[q], sem_in).wait()
      pltpu.async_copy(spm_h.at[idx_s[q]], val_v[q], sem_g)
      pltpu.make_async_copy(edges.at[pl.ds(0, CH)], idx_d[q], sem_in).wait()
    for q in range(U):
      pltpu.make_async_copy(spm_h.at[idx_s[q]], val_v[q], sem_g).wait()
      pltpu.async_copy(val_v[q], spm_agg.at[idx_d[q]], sem_w, add=True)
    for q in range(U):
      pltpu.make_async_copy(val_v[q], spm_agg.at[idx_d[q]], sem_w).wait()

    @pl.when(i + 1 < NI)
    def _():
      start_loads(i + 1)
    return 0
  lax.fori_loop(0, NI, chunk4, 0)

  plsc.subcore_barrier()
  pltpu.sync_copy(spm_agg.at[sl], agg_hbm.at[pl.ds(cid * NPAD + sid * SL, SL)])


_R4 = 1024
_G4 = NPAD // _R4


def _out_body(a0_ref, a1_ref, di0_ref, di1_ref, w_ref, out_ref):
  a = a0_ref[...] + a1_ref[...]
  d = di0_ref[...] + di1_ref[...]
  av = a * lax.rsqrt(jnp.maximum(d, 1.0))
  w128 = jnp.broadcast_to(w_ref[...], (128, 128))
  rr = lax.broadcasted_iota(jnp.int32, (128, 128), 0)
  cc = lax.broadcasted_iota(jnp.int32, (128, 128), 1)
  eye = rr == cc
  for s in range(_R4 // 128):
    m = jnp.broadcast_to(av[s:s + 1, :], (128, 128))
    dg = jnp.where(eye, m, 0.0)
    blk = lax.dot_general(dg, w128, (((1,), (0,)), ((), ())),
                          preferred_element_type=jnp.float32)
    out_ref[pl.ds(s * 128, 128), :] = blk


def kernel(read_length, edge_index, W):
  pad = (lax.iota(jnp.int32, 2 * EP) & 63) + N
  edges = pad.at[:E].set(edge_index[0]).at[EP:EP + E].set(edge_index[1])

  degout, degin = _hist_kernel(edges)
  degin = degin.reshape(NC, NPAD)

  rl = jnp.zeros((NPAD,), jnp.float32).at[:N].set(read_length)
  agg = _agg_kernel(edges, rl, degout).reshape(NC, NPAD)

  out = pl.pallas_call(
      _out_body,
      grid=(_G4,),
      in_specs=[
          pl.BlockSpec((_R4 // 128, 128), lambda g: (g, 0)),
          pl.BlockSpec((_R4 // 128, 128), lambda g: (g, 0)),
          pl.BlockSpec((_R4 // 128, 128), lambda g: (g, 0)),
          pl.BlockSpec((_R4 // 128, 128), lambda g: (g, 0)),
          pl.BlockSpec((1, 128), lambda g: (0, 0)),
      ],
      out_specs=pl.BlockSpec((_R4, 128), lambda g: (g, 0)),
      out_shape=jax.ShapeDtypeStruct((N, D), jnp.float32),
  )(agg[0].reshape(NPAD // 128, 128),
    agg[1].reshape(NPAD // 128, 128),
    degin[0].reshape(NPAD // 128, 128),
    degin[1].reshape(NPAD // 128, 128),
    W)
  return out


# speedup vs baseline: 124.5234x; 1.0206x over previous
import functools

import jax
import jax.numpy as jnp
from jax import lax
from jax.experimental import pallas as pl
from jax.experimental.pallas import tpu as pltpu
from jax.experimental.pallas import tpu_sc as plsc

N = 100000
E = 3200000
D = 128

NC = 2
NS = 16
NW = NC * NS

CH = 2048
CPW = 50
U = 2
NI = CPW // U
EP = NW * CPW * CH

NPAD = 100352
SL = NPAD // NS

assert CPW % U == 0 and NPAD % (NS * 8) == 0 and N + 64 <= NPAD and E <= EP


def _rsqrt16(d):
  d = jnp.maximum(d, 1.0)
  i = plsc.bitcast(d, jnp.int32)
  y = plsc.bitcast(0x5F3759DF - (i >> 1), jnp.float32)
  for _ in range(3):
    y = y * (1.5 - 0.5 * d * y * y)
  return y


_MESH = plsc.VectorSubcoreMesh(
    core_axis_name="c", subcore_axis_name="s", num_cores=NC, num_subcores=NS)

_SC_PARAMS = pltpu.CompilerParams(needs_layout_passes=False)


@functools.partial(
    pl.kernel,
    out_type=(
        jax.ShapeDtypeStruct((NC * NPAD,), jnp.float32),
        jax.ShapeDtypeStruct((NC * NPAD,), jnp.float32),
    ),
    mesh=_MESH,
    compiler_params=_SC_PARAMS,
    scratch_types=dict(
        idx_s=[pltpu.VMEM((CH,), jnp.int32) for _ in range(U)],
        idx_d=[pltpu.VMEM((CH,), jnp.int32) for _ in range(U)],
        ones_v=pltpu.VMEM((CH,), jnp.float32),
        zbuf=pltpu.VMEM((SL,), jnp.float32),
        spm_out=pltpu.VMEM_SHARED((NPAD,), jnp.float32),
        spm_in=pltpu.VMEM_SHARED((NPAD,), jnp.float32),
        sem_in=pltpu.SemaphoreType.DMA,
        sem_w=pltpu.SemaphoreType.DMA,
    ),
)
def _hist_kernel(edges, degout_hbm, degin_hbm,
                 idx_s, idx_d, ones_v, zbuf, spm_out, spm_in, sem_in, sem_w):
  cid = lax.axis_index("c")
  sid = lax.axis_index("s")
  w = cid * NS + sid

  def start_loads(j):
    for q in range(U):
      base = (w + (j * U + q) * NW) * CH
      pltpu.async_copy(edges.at[pl.ds(base, CH)], idx_s[q], sem_in)
      pltpu.async_copy(edges.at[pl.ds(EP + base, CH)], idx_d[q], sem_in)

  def wait_loads():
    for q in range(U):
      pltpu.make_async_copy(edges.at[pl.ds(0, CH)], idx_s[q], sem_in).wait()
      pltpu.make_async_copy(edges.at[pl.ds(0, CH)], idx_d[q], sem_in).wait()

  start_loads(0)

  def ob(i, _):
    ones_v[pl.ds(i * 16, 16)] = jnp.ones((16,), jnp.float32)
    return 0
  lax.fori_loop(0, CH // 16, ob, 0)

  def zb(i, _):
    zbuf[pl.ds(i * 16, 16)] = jnp.zeros((16,), jnp.float32)
    return 0
  lax.fori_loop(0, SL // 16, zb, 0)
  sl = pl.ds(sid * SL, SL)
  pltpu.sync_copy(zbuf, spm_out.at[sl])
  pltpu.sync_copy(zbuf, spm_in.at[sl])
  plsc.subcore_barrier()

  def chunk4(i, _):
    wait_loads()
    for q in range(U):
      pltpu.async_copy(ones_v, spm_out.at[idx_s[q]], sem_w, add=True)
      pltpu.async_copy(ones_v, spm_in.at[idx_d[q]], sem_w, add=True)
    for q in range(U):
      pltpu.make_async_copy(ones_v, spm_out.at[idx_s[q]], sem_w).wait()
      pltpu.make_async_copy(ones_v, spm_in.at[idx_d[q]], sem_w).wait()

    @pl.when(i + 1 < NI)
    def _():
      start_loads(i + 1)
    return 0
  lax.fori_loop(0, NI, chunk4, 0)

  plsc.subcore_barrier()
  osl = pl.ds(cid * NPAD + sid * SL, SL)
  pltpu.sync_copy(spm_out.at[sl], degout_hbm.at[osl])
  pltpu.sync_copy(spm_in.at[sl], degin_hbm.at[osl])


@functools.partial(
    pl.kernel,
    out_type=jax.ShapeDtypeStruct((NC * NPAD,), jnp.float32),
    mesh=_MESH,
    compiler_params=_SC_PARAMS,
    scratch_types=dict(
        idx_s=[pltpu.VMEM((CH,), jnp.int32) for _ in range(U)],
        idx_d=[pltpu.VMEM((CH,), jnp.int32) for _ in range(U)],
        val_v=[pltpu.VMEM((CH,), jnp.float32) for _ in range(U)],
        d0_v=pltpu.VMEM((SL,), jnp.float32),
        d1_v=pltpu.VMEM((SL,), jnp.float32),
        h_v=pltpu.VMEM((SL,), jnp.float32),
        spm_h=pltpu.VMEM_SHARED((NPAD,), jnp.float32),
        spm_agg=pltpu.VMEM_SHARED((NPAD,), jnp.float32),
        sem_in=pltpu.SemaphoreType.DMA,
        sem_g=pltpu.SemaphoreType.DMA,
        sem_w=pltpu.SemaphoreType.DMA,
    ),
)
def _agg_kernel(edges, rl_hbm, degout_hbm, agg_hbm,
                idx_s, idx_d, val_v, d0_v, d1_v, h_v,
                spm_h, spm_agg, sem_in, sem_g, sem_w):
  cid = lax.axis_index("c")
  sid = lax.axis_index("s")
  w = cid * NS + sid

  def start_loads(j):
    for q in range(U):
      base = (w + (j * U + q) * NW) * CH
      pltpu.async_copy(edges.at[pl.ds(base, CH)], idx_s[q], sem_in)
      pltpu.async_copy(edges.at[pl.ds(EP + base, CH)], idx_d[q], sem_in)

  start_loads(0)

  sl = pl.ds(sid * SL, SL)
  pltpu.sync_copy(degout_hbm.at[pl.ds(sid * SL, SL)], d0_v)
  pltpu.sync_copy(degout_hbm.at[pl.ds(NPAD + sid * SL, SL)], d1_v)
  pltpu.sync_copy(rl_hbm.at[sl], h_v)

  def hcomp(i, _):
    s16 = pl.ds(i * 16, 16)
    d = d0_v[s16] + d1_v[s16]
    h_v[s16] = h_v[s16] * (1.0 / 20000.0) * _rsqrt16(d)
    return 0
  lax.fori_loop(0, SL // 16, hcomp, 0)
  pltpu.sync_copy(h_v, spm_h.at[sl])

  def zb(i, _):
    d0_v[pl.ds(i * 16, 16)] = jnp.zeros((16,), jnp.float32)
    return 0
  lax.fori_loop(0, SL // 16, zb, 0)
  pltpu.sync_copy(d0_v, spm_agg.at[sl])
  plsc.subcore_barrier()

  def chunk4(i, _):
    for q in range(U):
      pltpu.make_async_copy(edges.at[pl.ds(0, CH)], idx_s[q], sem_in).wait()
      pltpu.async_copy(spm_h.at[idx_s[q]], val_v[q], sem_g)
      pltpu.make_async_copy(edges.at[pl.ds(0, CH)], idx_d[q], sem_in).wait()
    for q in range(U):
      pltpu.make_async_copy(spm_h.at[idx_s[q]], val_v[q], sem_g).wait()
      pltpu.async_copy(val_v[q], spm_agg.at[idx_d[q]], sem_w, add=True)
    for q in range(U):
      pltpu.make_async_copy(val_v[q], spm_agg.at[idx_d[q]], sem_w).wait()

    @pl.when(i + 1 < NI)
    def _():
      start_loads(i + 1)
    return 0
  lax.fori_loop(0, NI, chunk4, 0)

  plsc.subcore_barrier()
  pltpu.sync_copy(spm_agg.at[sl], agg_hbm.at[pl.ds(cid * NPAD + sid * SL, SL)])


_R4 = 1024
_G4 = NPAD // _R4


def _out_body(a0_ref, a1_ref, di0_ref, di1_ref, w_ref, out_ref):
  a = a0_ref[...] + a1_ref[...]
  d = di0_ref[...] + di1_ref[...]
  av = a * lax.rsqrt(jnp.maximum(d, 1.0))
  w128 = jnp.broadcast_to(w_ref[...], (128, 128))
  rr = lax.broadcasted_iota(jnp.int32, (128, 128), 0)
  cc = lax.broadcasted_iota(jnp.int32, (128, 128), 1)
  eye = rr == cc
  for s in range(_R4 // 128):
    m = jnp.broadcast_to(av[s:s + 1, :], (128, 128))
    dg = jnp.where(eye, m, 0.0)
    blk = lax.dot_general(dg, w128, (((1,), (0,)), ((), ())),
                          preferred_element_type=jnp.float32)
    out_ref[pl.ds(s * 128, 128), :] = blk


def kernel(read_length, edge_index, W):
  pad = (lax.iota(jnp.int32, 2 * EP) & 63) + N
  edges = pad.at[:E].set(edge_index[0]).at[EP:EP + E].set(edge_index[1])

  degout, degin = _hist_kernel(edges)
  degin = degin.reshape(NC, NPAD)

  rl = jnp.zeros((NPAD,), jnp.float32).at[:N].set(read_length)
  agg = _agg_kernel(edges, rl, degout).reshape(NC, NPAD)

  out = pl.pallas_call(
      _out_body,
      grid=(_G4,),
      in_specs=[
          pl.BlockSpec((_R4 // 128, 128), lambda g: (g, 0)),
          pl.BlockSpec((_R4 // 128, 128), lambda g: (g, 0)),
          pl.BlockSpec((_R4 // 128, 128), lambda g: (g, 0)),
          pl.BlockSpec((_R4 // 128, 128), lambda g: (g, 0)),
          pl.BlockSpec((1, 128), lambda g: (0, 0)),
      ],
      out_specs=pl.BlockSpec((_R4, 128), lambda g: (g, 0)),
      out_shape=jax.ShapeDtypeStruct((N, D), jnp.float32),
  )(agg[0].reshape(NPAD // 128, 128),
    agg[1].reshape(NPAD // 128, 128),
    degin[0].reshape(NPAD // 128, 128),
    degin[1].reshape(NPAD // 128, 128),
    W)
  return out


# speedup vs baseline: 138.0480x; 1.1086x over previous
import functools

import jax
import jax.numpy as jnp
from jax import lax
from jax.experimental import pallas as pl
from jax.experimental.pallas import tpu as pltpu
from jax.experimental.pallas import tpu_sc as plsc

N = 100000
E = 3200000
D = 128

NC = 2
NS = 16
NW = NC * NS

CH = 1024
NCH = E // CH
U = 4
NI = (NCH + NW * U - 1) // (NW * U)

NPAD = 100352
SL = NPAD // NS

assert E % CH == 0 and NPAD % (NS * 8) == 0 and N <= NPAD


def _rsqrt16(d):
  d = jnp.maximum(d, 1.0)
  i = plsc.bitcast(d, jnp.int32)
  y = plsc.bitcast(0x5F3759DF - (i >> 1), jnp.float32)
  for _ in range(3):
    y = y * (1.5 - 0.5 * d * y * y)
  return y


_MESH = plsc.VectorSubcoreMesh(
    core_axis_name="c", subcore_axis_name="s", num_cores=NC, num_subcores=NS)

_SC_PARAMS = pltpu.CompilerParams(
    needs_layout_passes=False, use_tc_tiling_on_sc=False)


@functools.partial(
    pl.kernel,
    out_type=(
        jax.ShapeDtypeStruct((NC * NPAD,), jnp.float32),
        jax.ShapeDtypeStruct((NC * NPAD,), jnp.float32),
    ),
    mesh=_MESH,
    compiler_params=_SC_PARAMS,
    scratch_types=dict(
        idx_s=[pltpu.VMEM((CH,), jnp.int32) for _ in range(U)],
        idx_d=[pltpu.VMEM((CH,), jnp.int32) for _ in range(U)],
        ones_v=pltpu.VMEM((CH,), jnp.float32),
        zbuf=pltpu.VMEM((SL,), jnp.float32),
        spm_out=pltpu.VMEM_SHARED((NPAD,), jnp.float32),
        spm_in=pltpu.VMEM_SHARED((NPAD,), jnp.float32),
        sem_in=pltpu.SemaphoreType.DMA,
        sem_w=pltpu.SemaphoreType.DMA,
    ),
)
def _hist_kernel(edges, degout_hbm, degin_hbm,
                 idx_s, idx_d, ones_v, zbuf, spm_out, spm_in, sem_in, sem_w):
  cid = lax.axis_index("c")
  sid = lax.axis_index("s")
  w = cid * NS + sid
  cnt = (NCH - w + NW - 1) // NW

  def start_loads(j):
    for q in range(U):
      ci = j * U + q
      base = (w + ci * NW) * CH

      @pl.when(ci < cnt)
      def _():
        pltpu.async_copy(edges.at[0, pl.ds(base, CH)], idx_s[q], sem_in)
        pltpu.async_copy(edges.at[1, pl.ds(base, CH)], idx_d[q], sem_in)

  start_loads(0)

  def ob(i, _):
    ones_v[pl.ds(i * 16, 16)] = jnp.ones((16,), jnp.float32)
    return 0
  lax.fori_loop(0, CH // 16, ob, 0)

  def zb(i, _):
    zbuf[pl.ds(i * 16, 16)] = jnp.zeros((16,), jnp.float32)
    return 0
  lax.fori_loop(0, SL // 16, zb, 0)
  sl = pl.ds(sid * SL, SL)
  pltpu.sync_copy(zbuf, spm_out.at[sl])
  pltpu.sync_copy(zbuf, spm_in.at[sl])
  plsc.subcore_barrier()

  def chunk4(i, _):
    for q in range(U):
      ci = i * U + q

      @pl.when(ci < cnt)
      def _():
        pltpu.make_async_copy(edges.at[0, pl.ds(0, CH)], idx_s[q],
                              sem_in).wait()
        pltpu.make_async_copy(edges.at[0, pl.ds(0, CH)], idx_d[q],
                              sem_in).wait()
        pltpu.async_copy(ones_v, spm_out.at[idx_s[q]], sem_w, add=True)
        pltpu.async_copy(ones_v, spm_in.at[idx_d[q]], sem_w, add=True)
    for q in range(U):
      ci = i * U + q

      @pl.when(ci < cnt)
      def _():
        pltpu.make_async_copy(ones_v, spm_out.at[idx_s[q]], sem_w).wait()
        pltpu.make_async_copy(ones_v, spm_in.at[idx_d[q]], sem_w).wait()

    @pl.when(i + 1 < NI)
    def _():
      start_loads(i + 1)
    return 0
  lax.fori_loop(0, NI, chunk4, 0)

  plsc.subcore_barrier()
  osl = pl.ds(cid * NPAD + sid * SL, SL)
  pltpu.sync_copy(spm_out.at[sl], degout_hbm.at[osl])
  pltpu.sync_copy(spm_in.at[sl], degin_hbm.at[osl])


@functools.partial(
    pl.kernel,
    out_type=jax.ShapeDtypeStruct((NC * NPAD,), jnp.float32),
    mesh=_MESH,
    compiler_params=_SC_PARAMS,
    scratch_types=dict(
        idx_s=[pltpu.VMEM((CH,), jnp.int32) for _ in range(U)],
        idx_d=[pltpu.VMEM((CH,), jnp.int32) for _ in range(U)],
        val_v=[pltpu.VMEM((CH,), jnp.float32) for _ in range(U)],
        d0_v=pltpu.VMEM((SL,), jnp.float32),
        d1_v=pltpu.VMEM((SL,), jnp.float32),
        h_v=pltpu.VMEM((SL,), jnp.float32),
        spm_h=pltpu.VMEM_SHARED((NPAD,), jnp.float32),
        spm_agg=pltpu.VMEM_SHARED((NPAD,), jnp.float32),
        sem_in=pltpu.SemaphoreType.DMA,
        sem_g=pltpu.SemaphoreType.DMA,
        sem_w=pltpu.SemaphoreType.DMA,
    ),
)
def _agg_kernel(edges, rl_hbm, degout_hbm, agg_hbm,
                idx_s, idx_d, val_v, d0_v, d1_v, h_v,
                spm_h, spm_agg, sem_in, sem_g, sem_w):
  cid = lax.axis_index("c")
  sid = lax.axis_index("s")
  w = cid * NS + sid
  cnt = (NCH - w + NW - 1) // NW

  def start_loads(j):
    for q in range(U):
      ci = j * U + q
      base = (w + ci * NW) * CH

      @pl.when(ci < cnt)
      def _():
        pltpu.async_copy(edges.at[0, pl.ds(base, CH)], idx_s[q], sem_in)
        pltpu.async_copy(edges.at[1, pl.ds(base, CH)], idx_d[q], sem_in)

  start_loads(0)

  sl = pl.ds(sid * SL, SL)
  pltpu.sync_copy(degout_hbm.at[pl.ds(sid * SL, SL)], d0_v)
  pltpu.sync_copy(degout_hbm.at[pl.ds(NPAD + sid * SL, SL)], d1_v)
  pltpu.sync_copy(rl_hbm.at[sl], h_v)

  def hcomp(i, _):
    s16 = pl.ds(i * 16, 16)
    d = d0_v[s16] + d1_v[s16]
    h_v[s16] = h_v[s16] * (1.0 / 20000.0) * _rsqrt16(d)
    return 0
  lax.fori_loop(0, SL // 16, hcomp, 0)
  pltpu.sync_copy(h_v, spm_h.at[sl])

  def zb(i, _):
    d0_v[pl.ds(i * 16, 16)] = jnp.zeros((16,), jnp.float32)
    return 0
  lax.fori_loop(0, SL // 16, zb, 0)
  pltpu.sync_copy(d0_v, spm_agg.at[sl])
  plsc.subcore_barrier()

  def chunk4(i, _):
    for q in range(U):
      ci = i * U + q

      @pl.when(ci < cnt)
      def _():
        pltpu.make_async_copy(edges.at[0, pl.ds(0, CH)], idx_s[q],
                              sem_in).wait()
        pltpu.async_copy(spm_h.at[idx_s[q]], val_v[q], sem_g)
        pltpu.make_async_copy(edges.at[0, pl.ds(0, CH)], idx_d[q],
                              sem_in).wait()
    for q in range(U):
      ci = i * U + q

      @pl.when(ci < cnt)
      def _():
        pltpu.make_async_copy(spm_h.at[idx_s[q]], val_v[q], sem_g).wait()
        pltpu.async_copy(val_v[q], spm_agg.at[idx_d[q]], sem_w, add=True)
    for q in range(U):
      ci = i * U + q

      @pl.when(ci < cnt)
      def _():
        pltpu.make_async_copy(val_v[q], spm_agg.at[idx_d[q]], sem_w).wait()

    @pl.when(i + 1 < NI)
    def _():
      start_loads(i + 1)
    return 0
  lax.fori_loop(0, NI, chunk4, 0)

  plsc.subcore_barrier()
  pltpu.sync_copy(spm_agg.at[sl], agg_hbm.at[pl.ds(cid * NPAD + sid * SL, SL)])


_R4 = 1024
_G4 = NPAD // _R4


def _out_body(a0_ref, a1_ref, di0_ref, di1_ref, w_ref, out_ref):
  a = a0_ref[...] + a1_ref[...]
  d = di0_ref[...] + di1_ref[...]
  av = a * lax.rsqrt(jnp.maximum(d, 1.0))
  w128 = jnp.broadcast_to(w_ref[...], (128, 128))
  rr = lax.broadcasted_iota(jnp.int32, (128, 128), 0)
  cc = lax.broadcasted_iota(jnp.int32, (128, 128), 1)
  eye = rr == cc
  for s in range(_R4 // 128):
    m = jnp.broadcast_to(av[s:s + 1, :], (128, 128))
    dg = jnp.where(eye, m, 0.0)
    blk = lax.dot_general(dg, w128, (((1,), (0,)), ((), ())),
                          preferred_element_type=jnp.float32)
    out_ref[pl.ds(s * 128, 128), :] = blk


def kernel(read_length, edge_index, W):
  degout, degin = _hist_kernel(edge_index)
  degin = degin.reshape(NC, NPAD)

  rl = jnp.zeros((NPAD,), jnp.float32).at[:N].set(read_length)
  agg = _agg_kernel(edge_index, rl, degout).reshape(NC, NPAD)

  out = pl.pallas_call(
      _out_body,
      grid=(_G4,),
      in_specs=[
          pl.BlockSpec((_R4 // 128, 128), lambda g: (g, 0)),
          pl.BlockSpec((_R4 // 128, 128), lambda g: (g, 0)),
          pl.BlockSpec((_R4 // 128, 128), lambda g: (g, 0)),
          pl.BlockSpec((_R4 // 128, 128), lambda g: (g, 0)),
          pl.BlockSpec((1, 128), lambda g: (0, 0)),
      ],
      out_specs=pl.BlockSpec((_R4, 128), lambda g: (g, 0)),
      out_shape=jax.ShapeDtypeStruct((N, D), jnp.float32),
  )(agg[0].reshape(NPAD // 128, 128),
    agg[1].reshape(NPAD // 128, 128),
    degin[0].reshape(NPAD // 128, 128),
    degin[1].reshape(NPAD // 128, 128),
    W)
  return out


# speedup vs baseline: 140.3485x; 1.0167x over previous
import functools

import jax
import jax.numpy as jnp
from jax import lax
from jax.experimental import pallas as pl
from jax.experimental.pallas import tpu as pltpu
from jax.experimental.pallas import tpu_sc as plsc

N = 100000
E = 3200000
D = 128

NC = 2
NS = 16
NW = NC * NS

CH = 1024
NCH = E // CH
U = 4
NI = (NCH + NW * U - 1) // (NW * U)

NPAD = 100352
SL = NPAD // NS

assert E % CH == 0 and NPAD % (NS * 8) == 0 and N <= NPAD


def _rsqrt16(d):
  d = jnp.maximum(d, 1.0)
  i = plsc.bitcast(d, jnp.int32)
  y = plsc.bitcast(0x5F3759DF - (i >> 1), jnp.float32)
  for _ in range(3):
    y = y * (1.5 - 0.5 * d * y * y)
  return y


_MESH = plsc.VectorSubcoreMesh(
    core_axis_name="c", subcore_axis_name="s", num_cores=NC, num_subcores=NS)

_SC_PARAMS = pltpu.CompilerParams(
    needs_layout_passes=False, use_tc_tiling_on_sc=False)


@functools.partial(
    pl.kernel,
    out_type=(
        jax.ShapeDtypeStruct((NC * NPAD,), jnp.float32),
        jax.ShapeDtypeStruct((NC * NPAD,), jnp.float32),
    ),
    mesh=_MESH,
    compiler_params=_SC_PARAMS,
    scratch_types=dict(
        idx_s=[pltpu.VMEM((CH,), jnp.int32) for _ in range(U)],
        idx_d=[pltpu.VMEM((CH,), jnp.int32) for _ in range(U)],
        ones_v=pltpu.VMEM((CH,), jnp.float32),
        zbuf=pltpu.VMEM((SL,), jnp.float32),
        spm_out=pltpu.VMEM_SHARED((NPAD,), jnp.float32),
        spm_in=pltpu.VMEM_SHARED((NPAD,), jnp.float32),
        sem_in=pltpu.SemaphoreType.DMA,
        sem_w=pltpu.SemaphoreType.DMA,
    ),
)
def _hist_kernel(edges, degout_hbm, degin_hbm,
                 idx_s, idx_d, ones_v, zbuf, spm_out, spm_in, sem_in, sem_w):
  cid = lax.axis_index("c")
  sid = lax.axis_index("s")
  w = cid * NS + sid
  cnt = (NCH - w + NW - 1) // NW

  def start_loads(j):
    for q in range(U):
      ci = j * U + q
      base = (w + ci * NW) * CH

      @pl.when(ci < cnt)
      def _():
        pltpu.async_copy(edges.at[0, pl.ds(base, CH)], idx_s[q], sem_in)
        pltpu.async_copy(edges.at[1, pl.ds(base, CH)], idx_d[q], sem_in)

  start_loads(0)

  def ob(i, _):
    ones_v[pl.ds(i * 16, 16)] = jnp.ones((16,), jnp.float32)
    return 0
  lax.fori_loop(0, CH // 16, ob, 0)

  def zb(i, _):
    zbuf[pl.ds(i * 16, 16)] = jnp.zeros((16,), jnp.float32)
    return 0
  lax.fori_loop(0, SL // 16, zb, 0)
  sl = pl.ds(sid * SL, SL)
  pltpu.sync_copy(zbuf, spm_out.at[sl])
  pltpu.sync_copy(zbuf, spm_in.at[sl])
  plsc.subcore_barrier()

  def chunk4(i, _):
    for q in range(U):
      ci = i * U + q

      @pl.when(ci < cnt)
      def _():
        pltpu.make_async_copy(edges.at[0, pl.ds(0, CH)], idx_s[q],
                              sem_in).wait()
        pltpu.make_async_copy(edges.at[0, pl.ds(0, CH)], idx_d[q],
                              sem_in).wait()
        pltpu.async_copy(ones_v, spm_out.at[idx_s[q]], sem_w, add=True)
        pltpu.async_copy(ones_v, spm_in.at[idx_d[q]], sem_w, add=True)
    for q in range(U):
      ci = i * U + q

      @pl.when(ci < cnt)
      def _():
        pltpu.make_async_copy(ones_v, spm_out.at[idx_s[q]], sem_w).wait()
        pltpu.make_async_copy(ones_v, spm_in.at[idx_d[q]], sem_w).wait()

    @pl.when(i + 1 < NI)
    def _():
      start_loads(i + 1)
    return 0
  lax.fori_loop(0, NI, chunk4, 0)

  plsc.subcore_barrier()
  osl = pl.ds(cid * NPAD + sid * SL, SL)
  pltpu.sync_copy(spm_out.at[sl], degout_hbm.at[osl])
  pltpu.sync_copy(spm_in.at[sl], degin_hbm.at[osl])


@functools.partial(
    pl.kernel,
    out_type=jax.ShapeDtypeStruct((NC * NPAD,), jnp.float32),
    mesh=_MESH,
    compiler_params=_SC_PARAMS,
    scratch_types=dict(
        idx_s=[pltpu.VMEM((CH,), jnp.int32) for _ in range(U)],
        idx_d=[pltpu.VMEM((CH,), jnp.int32) for _ in range(U)],
        val_v=[pltpu.VMEM((CH,), jnp.float32) for _ in range(U)],
        d0_v=pltpu.VMEM((SL,), jnp.float32),
        d1_v=pltpu.VMEM((SL,), jnp.float32),
        h_v=pltpu.VMEM((SL,), jnp.float32),
        spm_h=pltpu.VMEM_SHARED((NPAD,), jnp.float32),
        spm_agg=pltpu.VMEM_SHARED((NPAD,), jnp.float32),
        sem_in=pltpu.SemaphoreType.DMA,
        sem_g=pltpu.SemaphoreType.DMA,
        sem_w=pltpu.SemaphoreType.DMA,
    ),
)
def _agg_kernel(edges, rl_hbm, degout_hbm, agg_hbm,
                idx_s, idx_d, val_v, d0_v, d1_v, h_v,
                spm_h, spm_agg, sem_in, sem_g, sem_w):
  cid = lax.axis_index("c")
  sid = lax.axis_index("s")
  w = cid * NS + sid
  cnt = (NCH - w + NW - 1) // NW

  def start_loads(j):
    for q in range(U):
      ci = j * U + q
      base = (w + ci * NW) * CH

      @pl.when(ci < cnt)
      def _():
        pltpu.async_copy(edges.at[0, pl.ds(base, CH)], idx_s[q], sem_in)
        pltpu.async_copy(edges.at[1, pl.ds(base, CH)], idx_d[q], sem_in)

  start_loads(0)

  sl = pl.ds(sid * SL, SL)
  pltpu.sync_copy(degout_hbm.at[pl.ds(sid * SL, SL)], d0_v)
  pltpu.sync_copy(degout_hbm.at[pl.ds(NPAD + sid * SL, SL)], d1_v)
  pltpu.sync_copy(rl_hbm.at[sl], h_v)

  def hcomp(i, _):
    s16 = pl.ds(i * 16, 16)
    d = d0_v[s16] + d1_v[s16]
    h_v[s16] = h_v[s16] * (1.0 / 20000.0) * _rsqrt16(d)
    return 0
  lax.fori_loop(0, SL // 16, hcomp, 0)
  pltpu.sync_copy(h_v, spm_h.at[sl])

  def zb(i, _):
    d0_v[pl.ds(i * 16, 16)] = jnp.zeros((16,), jnp.float32)
    return 0
  lax.fori_loop(0, SL // 16, zb, 0)
  pltpu.sync_copy(d0_v, spm_agg.at[sl])
  plsc.subcore_barrier()

  def chunk4(i, _):
    for q in range(U):
      ci = i * U + q

      @pl.when(ci < cnt)
      def _():
        pltpu.make_async_copy(edges.at[0, pl.ds(0, CH)], idx_s[q],
                              sem_in).wait()
        pltpu.async_copy(spm_h.at[idx_s[q]], val_v[q], sem_g)
        pltpu.make_async_copy(edges.at[0, pl.ds(0, CH)], idx_d[q],
                              sem_in).wait()
    for q in range(U):
      ci = i * U + q

      @pl.when(ci < cnt)
      def _():
        pltpu.make_async_copy(spm_h.at[idx_s[q]], val_v[q], sem_g).wait()
        pltpu.async_copy(val_v[q], spm_agg.at[idx_d[q]], sem_w, add=True)
    for q in range(U):
      ci = i * U + q

      @pl.when(ci < cnt)
      def _():
        pltpu.make_async_copy(val_v[q], spm_agg.at[idx_d[q]],
                              sem_w).wait()

    @pl.when(i + 1 < NI)
    def _():
      start_loads(i + 1)
    return 0
  lax.fori_loop(0, NI, chunk4, 0)

  plsc.subcore_barrier()
  pltpu.sync_copy(spm_agg.at[sl], agg_hbm.at[pl.ds(cid * NPAD + sid * SL, SL)])


_R4 = 1024
_G4 = NPAD // _R4
_HB = NPAD // 128


def _out_body(a0_ref, a1_ref, di0_ref, di1_ref, w_ref, out_ref):
  a = a0_ref[...] + a1_ref[...]
  d = di0_ref[...] + di1_ref[...]
  av = a * lax.rsqrt(jnp.maximum(d, 1.0))
  w128 = jnp.broadcast_to(w_ref[...], (128, 128))
  rr = lax.broadcasted_iota(jnp.int32, (128, 128), 0)
  cc = lax.broadcasted_iota(jnp.int32, (128, 128), 1)
  eye = rr == cc
  for s in range(_R4 // 128):
    m = jnp.broadcast_to(av[s:s + 1, :], (128, 128))
    dg = jnp.where(eye, m, 0.0)
    blk = lax.dot_general(dg, w128, (((1,), (0,)), ((), ())),
                          preferred_element_type=jnp.float32)
    out_ref[pl.ds(s * 128, 128), :] = blk


def kernel(read_length, edge_index, W):
  degout, degin = _hist_kernel(edge_index)

  rl = jnp.zeros((NPAD,), jnp.float32).at[:N].set(read_length)
  agg = _agg_kernel(edge_index, rl, degout).reshape(2 * _HB, 128)
  degin = degin.reshape(2 * _HB, 128)

  blk8 = _R4 // 128
  out = pl.pallas_call(
      _out_body,
      grid=(_G4,),
      in_specs=[
          pl.BlockSpec((blk8, 128), lambda g: (g, 0)),
          pl.BlockSpec((blk8, 128), lambda g: (g + _HB // blk8, 0)),
          pl.BlockSpec((blk8, 128), lambda g: (g, 0)),
          pl.BlockSpec((blk8, 128), lambda g: (g + _HB // blk8, 0)),
          pl.BlockSpec((1, 128), lambda g: (0, 0)),
      ],
      out_specs=pl.BlockSpec((_R4, 128), lambda g: (g, 0)),
      out_shape=jax.ShapeDtypeStruct((N, D), jnp.float32),
  )(agg, agg, degin, degin, W)
  return out


# speedup vs baseline: 144.3496x; 1.0285x over previous
import functools

import jax
import jax.numpy as jnp
from jax import lax
from jax.experimental import pallas as pl
from jax.experimental.pallas import tpu as pltpu
from jax.experimental.pallas import tpu_sc as plsc

N = 100000
E = 3200000
D = 128

NC = 2
NS = 16
NW = NC * NS

CH = 1024
NCH = E // CH
U = 8
NI = (NCH + NW * U - 1) // (NW * U)

NPAD = 100352
SL = NPAD // NS

assert E % CH == 0 and NPAD % (NS * 8) == 0 and N <= NPAD


def _rsqrt16(d):
  d = jnp.maximum(d, 1.0)
  i = plsc.bitcast(d, jnp.int32)
  y = plsc.bitcast(0x5F3759DF - (i >> 1), jnp.float32)
  for _ in range(3):
    y = y * (1.5 - 0.5 * d * y * y)
  return y


_MESH = plsc.VectorSubcoreMesh(
    core_axis_name="c", subcore_axis_name="s", num_cores=NC, num_subcores=NS)

_SC_PARAMS = pltpu.CompilerParams(
    needs_layout_passes=False, use_tc_tiling_on_sc=False)


@functools.partial(
    pl.kernel,
    out_type=(
        jax.ShapeDtypeStruct((NC * NPAD,), jnp.float32),
        jax.ShapeDtypeStruct((NC * NPAD,), jnp.float32),
    ),
    mesh=_MESH,
    compiler_params=_SC_PARAMS,
    scratch_types=dict(
        idx_s=[pltpu.VMEM((CH,), jnp.int32) for _ in range(U)],
        idx_d=[pltpu.VMEM((CH,), jnp.int32) for _ in range(U)],
        ones_v=pltpu.VMEM((CH,), jnp.float32),
        zbuf=pltpu.VMEM((SL,), jnp.float32),
        spm_out=pltpu.VMEM_SHARED((NPAD,), jnp.float32),
        spm_in=pltpu.VMEM_SHARED((NPAD,), jnp.float32),
        sem_in=pltpu.SemaphoreType.DMA,
        sem_w=pltpu.SemaphoreType.DMA,
    ),
)
def _hist_kernel(edges, degout_hbm, degin_hbm,
                 idx_s, idx_d, ones_v, zbuf, spm_out, spm_in, sem_in, sem_w):
  cid = lax.axis_index("c")
  sid = lax.axis_index("s")
  w = cid * NS + sid
  cnt = (NCH - w + NW - 1) // NW

  def start_loads(j):
    for q in range(U):
      ci = j * U + q
      base = (w + ci * NW) * CH

      @pl.when(ci < cnt)
      def _():
        pltpu.async_copy(edges.at[0, pl.ds(base, CH)], idx_s[q], sem_in)
        pltpu.async_copy(edges.at[1, pl.ds(base, CH)], idx_d[q], sem_in)

  start_loads(0)

  def ob(i, _):
    ones_v[pl.ds(i * 16, 16)] = jnp.ones((16,), jnp.float32)
    return 0
  lax.fori_loop(0, CH // 16, ob, 0)

  def zb(i, _):
    zbuf[pl.ds(i * 16, 16)] = jnp.zeros((16,), jnp.float32)
    return 0
  lax.fori_loop(0, SL // 16, zb, 0)
  sl = pl.ds(sid * SL, SL)
  pltpu.sync_copy(zbuf, spm_out.at[sl])
  pltpu.sync_copy(zbuf, spm_in.at[sl])
  plsc.subcore_barrier()

  def chunk4(i, _):
    for q in range(U):
      ci = i * U + q

      @pl.when(ci < cnt)
      def _():
        pltpu.make_async_copy(edges.at[0, pl.ds(0, CH)], idx_s[q],
                              sem_in).wait()
        pltpu.make_async_copy(edges.at[0, pl.ds(0, CH)], idx_d[q],
                              sem_in).wait()
        pltpu.async_copy(ones_v, spm_out.at[idx_s[q]], sem_w, add=True)
        pltpu.async_copy(ones_v, spm_in.at[idx_d[q]], sem_w, add=True)
    for q in range(U):
      ci = i * U + q

      @pl.when(ci < cnt)
      def _():
        pltpu.make_async_copy(ones_v, spm_out.at[idx_s[q]], sem_w).wait()
        pltpu.make_async_copy(ones_v, spm_in.at[idx_d[q]], sem_w).wait()

    @pl.when(i + 1 < NI)
    def _():
      start_loads(i + 1)
    return 0
  lax.fori_loop(0, NI, chunk4, 0)

  plsc.subcore_barrier()
  osl = pl.ds(cid * NPAD + sid * SL, SL)
  pltpu.sync_copy(spm_out.at[sl], degout_hbm.at[osl])
  pltpu.sync_copy(spm_in.at[sl], degin_hbm.at[osl])


@functools.partial(
    pl.kernel,
    out_type=jax.ShapeDtypeStruct((NC * NPAD,), jnp.float32),
    mesh=_MESH,
    compiler_params=_SC_PARAMS,
    scratch_types=dict(
        idx_s=[pltpu.VMEM((CH,), jnp.int32) for _ in range(U)],
        idx_d=[pltpu.VMEM((CH,), jnp.int32) for _ in range(U)],
        val_v=[pltpu.VMEM((CH,), jnp.float32) for _ in range(U)],
        d0_v=pltpu.VMEM((SL,), jnp.float32),
        d1_v=pltpu.VMEM((SL,), jnp.float32),
        h_v=pltpu.VMEM((SL,), jnp.float32),
        spm_h=pltpu.VMEM_SHARED((NPAD,), jnp.float32),
        spm_agg=pltpu.VMEM_SHARED((NPAD,), jnp.float32),
        sem_in=pltpu.SemaphoreType.DMA,
        sem_g=pltpu.SemaphoreType.DMA,
        sem_w=pltpu.SemaphoreType.DMA,
    ),
)
def _agg_kernel(edges, rl_hbm, degout_hbm, agg_hbm,
                idx_s, idx_d, val_v, d0_v, d1_v, h_v,
                spm_h, spm_agg, sem_in, sem_g, sem_w):
  cid = lax.axis_index("c")
  sid = lax.axis_index("s")
  w = cid * NS + sid
  cnt = (NCH - w + NW - 1) // NW

  def start_loads(j):
    for q in range(U):
      ci = j * U + q
      base = (w + ci * NW) * CH

      @pl.when(ci < cnt)
      def _():
        pltpu.async_copy(edges.at[0, pl.ds(base, CH)], idx_s[q], sem_in)
        pltpu.async_copy(edges.at[1, pl.ds(base, CH)], idx_d[q], sem_in)

  start_loads(0)

  sl = pl.ds(sid * SL, SL)
  pltpu.sync_copy(degout_hbm.at[pl.ds(sid * SL, SL)], d0_v)
  pltpu.sync_copy(degout_hbm.at[pl.ds(NPAD + sid * SL, SL)], d1_v)
  pltpu.sync_copy(rl_hbm.at[sl], h_v)

  def hcomp(i, _):
    s16 = pl.ds(i * 16, 16)
    d = d0_v[s16] + d1_v[s16]
    h_v[s16] = h_v[s16] * (1.0 / 20000.0) * _rsqrt16(d)
    return 0
  lax.fori_loop(0, SL // 16, hcomp, 0)
  pltpu.sync_copy(h_v, spm_h.at[sl])

  def zb(i, _):
    d0_v[pl.ds(i * 16, 16)] = jnp.zeros((16,), jnp.float32)
    return 0
  lax.fori_loop(0, SL // 16, zb, 0)
  pltpu.sync_copy(d0_v, spm_agg.at[sl])
  plsc.subcore_barrier()

  def chunk4(i, _):
    for q in range(U):
      ci = i * U + q

      @pl.when(ci < cnt)
      def _():
        pltpu.make_async_copy(edges.at[0, pl.ds(0, CH)], idx_s[q],
                              sem_in).wait()
        pltpu.async_copy(spm_h.at[idx_s[q]], val_v[q], sem_g)
        pltpu.make_async_copy(edges.at[0, pl.ds(0, CH)], idx_d[q],
                              sem_in).wait()
    for q in range(U):
      ci = i * U + q

      @pl.when(ci < cnt)
      def _():
        pltpu.make_async_copy(spm_h.at[idx_s[q]], val_v[q], sem_g).wait()
        pltpu.async_copy(val_v[q], spm_agg.at[idx_d[q]], sem_w, add=True)
    for q in range(U):
      ci = i * U + q

      @pl.when(ci < cnt)
      def _():
        pltpu.make_async_copy(val_v[q], spm_agg.at[idx_d[q]],
                              sem_w).wait()

    @pl.when(i + 1 < NI)
    def _():
      start_loads(i + 1)
    return 0
  lax.fori_loop(0, NI, chunk4, 0)

  plsc.subcore_barrier()
  pltpu.sync_copy(spm_agg.at[sl], agg_hbm.at[pl.ds(cid * NPAD + sid * SL, SL)])


_R4 = 1024
_G4 = NPAD // _R4
_HB = NPAD // 128


def _out_body(a0_ref, a1_ref, di0_ref, di1_ref, w_ref, out_ref):
  a = a0_ref[...] + a1_ref[...]
  d = di0_ref[...] + di1_ref[...]
  av = a * lax.rsqrt(jnp.maximum(d, 1.0))
  w128 = jnp.broadcast_to(w_ref[...], (128, 128))
  rr = lax.broadcasted_iota(jnp.int32, (128, 128), 0)
  cc = lax.broadcasted_iota(jnp.int32, (128, 128), 1)
  eye = rr == cc
  for s in range(_R4 // 128):
    m = jnp.broadcast_to(av[s:s + 1, :], (128, 128))
    dg = jnp.where(eye, m, 0.0)
    blk = lax.dot_general(dg, w128, (((1,), (0,)), ((), ())),
                          preferred_element_type=jnp.float32)
    out_ref[pl.ds(s * 128, 128), :] = blk


def kernel(read_length, edge_index, W):
  degout, degin = _hist_kernel(edge_index)

  rl = jnp.zeros((NPAD,), jnp.float32).at[:N].set(read_length)
  agg = _agg_kernel(edge_index, rl, degout).reshape(2 * _HB, 128)
  degin = degin.reshape(2 * _HB, 128)

  blk8 = _R4 // 128
  out = pl.pallas_call(
      _out_body,
      grid=(_G4,),
      in_specs=[
          pl.BlockSpec((blk8, 128), lambda g: (g, 0)),
          pl.BlockSpec((blk8, 128), lambda g: (g + _HB // blk8, 0)),
          pl.BlockSpec((blk8, 128), lambda g: (g, 0)),
          pl.BlockSpec((blk8, 128), lambda g: (g + _HB // blk8, 0)),
          pl.BlockSpec((1, 128), lambda g: (0, 0)),
      ],
      out_specs=pl.BlockSpec((_R4, 128), lambda g: (g, 0)),
      out_shape=jax.ShapeDtypeStruct((N, D), jnp.float32),
  )(agg, agg, degin, degin, W)
  return out


# speedup vs baseline: 151.7081x; 1.0510x over previous
import functools

import jax
import jax.numpy as jnp
from jax import lax
from jax.experimental import pallas as pl
from jax.experimental.pallas import tpu as pltpu
from jax.experimental.pallas import tpu_sc as plsc

N = 100000
E = 3200000
D = 128

NC = 2
NS = 16
NW = NC * NS

CH = 1024
NCH = E // CH
U = 14
NI = (NCH + NW * U - 1) // (NW * U)

NPAD = 100352
SL = NPAD // NS

assert E % CH == 0 and NPAD % (NS * 8) == 0 and N <= NPAD


def _rsqrt16(d):
  d = jnp.maximum(d, 1.0)
  i = plsc.bitcast(d, jnp.int32)
  y = plsc.bitcast(0x5F3759DF - (i >> 1), jnp.float32)
  for _ in range(3):
    y = y * (1.5 - 0.5 * d * y * y)
  return y


_MESH = plsc.VectorSubcoreMesh(
    core_axis_name="c", subcore_axis_name="s", num_cores=NC, num_subcores=NS)

_SC_PARAMS = pltpu.CompilerParams(
    needs_layout_passes=False, use_tc_tiling_on_sc=False)


@functools.partial(
    pl.kernel,
    out_type=(
        jax.ShapeDtypeStruct((NC * NPAD,), jnp.float32),
        jax.ShapeDtypeStruct((NC * NPAD,), jnp.float32),
    ),
    mesh=_MESH,
    compiler_params=_SC_PARAMS,
    scratch_types=dict(
        idx_s=[pltpu.VMEM((CH,), jnp.int32) for _ in range(U)],
        idx_d=[pltpu.VMEM((CH,), jnp.int32) for _ in range(U)],
        ones_v=pltpu.VMEM((CH,), jnp.float32),
        zbuf=pltpu.VMEM((SL,), jnp.float32),
        spm_out=pltpu.VMEM_SHARED((NPAD,), jnp.float32),
        spm_in=pltpu.VMEM_SHARED((NPAD,), jnp.float32),
        sem_in=pltpu.SemaphoreType.DMA,
        sem_w=pltpu.SemaphoreType.DMA,
    ),
)
def _hist_kernel(edges, degout_hbm, degin_hbm,
                 idx_s, idx_d, ones_v, zbuf, spm_out, spm_in, sem_in, sem_w):
  cid = lax.axis_index("c")
  sid = lax.axis_index("s")
  w = cid * NS + sid
  cnt = (NCH - w + NW - 1) // NW

  def start_loads(j):
    for q in range(U):
      ci = j * U + q
      base = (w + ci * NW) * CH

      @pl.when(ci < cnt)
      def _():
        pltpu.async_copy(edges.at[0, pl.ds(base, CH)], idx_s[q], sem_in)
        pltpu.async_copy(edges.at[1, pl.ds(base, CH)], idx_d[q], sem_in)

  start_loads(0)

  def ob(i, _):
    ones_v[pl.ds(i * 16, 16)] = jnp.ones((16,), jnp.float32)
    return 0
  lax.fori_loop(0, CH // 16, ob, 0)

  def zb(i, _):
    zbuf[pl.ds(i * 16, 16)] = jnp.zeros((16,), jnp.float32)
    return 0
  lax.fori_loop(0, SL // 16, zb, 0)
  sl = pl.ds(sid * SL, SL)
  pltpu.sync_copy(zbuf, spm_out.at[sl])
  pltpu.sync_copy(zbuf, spm_in.at[sl])
  plsc.subcore_barrier()

  def chunk4(i, _):
    for q in range(U):
      ci = i * U + q

      @pl.when(ci < cnt)
      def _():
        pltpu.make_async_copy(edges.at[0, pl.ds(0, CH)], idx_s[q],
                              sem_in).wait()
        pltpu.make_async_copy(edges.at[0, pl.ds(0, CH)], idx_d[q],
                              sem_in).wait()
        pltpu.async_copy(ones_v, spm_out.at[idx_s[q]], sem_w, add=True)
        pltpu.async_copy(ones_v, spm_in.at[idx_d[q]], sem_w, add=True)
    for q in range(U):
      ci = i * U + q

      @pl.when(ci < cnt)
      def _():
        pltpu.make_async_copy(ones_v, spm_out.at[idx_s[q]], sem_w).wait()
        pltpu.make_async_copy(ones_v, spm_in.at[idx_d[q]], sem_w).wait()

    @pl.when(i + 1 < NI)
    def _():
      start_loads(i + 1)
    return 0
  lax.fori_loop(0, NI, chunk4, 0)

  plsc.subcore_barrier()
  osl = pl.ds(cid * NPAD + sid * SL, SL)
  pltpu.sync_copy(spm_out.at[sl], degout_hbm.at[osl])
  pltpu.sync_copy(spm_in.at[sl], degin_hbm.at[osl])


@functools.partial(
    pl.kernel,
    out_type=jax.ShapeDtypeStruct((NC * NPAD,), jnp.float32),
    mesh=_MESH,
    compiler_params=_SC_PARAMS,
    scratch_types=dict(
        idx_s=[pltpu.VMEM((CH,), jnp.int32) for _ in range(U)],
        idx_d=[pltpu.VMEM((CH,), jnp.int32) for _ in range(U)],
        val_v=[pltpu.VMEM((CH,), jnp.float32) for _ in range(U)],
        d0_v=pltpu.VMEM((SL,), jnp.float32),
        d1_v=pltpu.VMEM((SL,), jnp.float32),
        h_v=pltpu.VMEM((SL,), jnp.float32),
        spm_h=pltpu.VMEM_SHARED((NPAD,), jnp.float32),
        spm_agg=pltpu.VMEM_SHARED((NPAD,), jnp.float32),
        sem_in=pltpu.SemaphoreType.DMA,
        sem_g=pltpu.SemaphoreType.DMA,
        sem_w=pltpu.SemaphoreType.DMA,
    ),
)
def _agg_kernel(edges, rl_hbm, degout_hbm, agg_hbm,
                idx_s, idx_d, val_v, d0_v, d1_v, h_v,
                spm_h, spm_agg, sem_in, sem_g, sem_w):
  cid = lax.axis_index("c")
  sid = lax.axis_index("s")
  w = cid * NS + sid
  cnt = (NCH - w + NW - 1) // NW

  def start_loads(j):
    for q in range(U):
      ci = j * U + q
      base = (w + ci * NW) * CH

      @pl.when(ci < cnt)
      def _():
        pltpu.async_copy(edges.at[0, pl.ds(base, CH)], idx_s[q], sem_in)
        pltpu.async_copy(edges.at[1, pl.ds(base, CH)], idx_d[q], sem_in)

  start_loads(0)

  sl = pl.ds(sid * SL, SL)
  pltpu.sync_copy(degout_hbm.at[pl.ds(sid * SL, SL)], d0_v)
  pltpu.sync_copy(degout_hbm.at[pl.ds(NPAD + sid * SL, SL)], d1_v)
  pltpu.sync_copy(rl_hbm.at[sl], h_v)

  def hcomp(i, _):
    s16 = pl.ds(i * 16, 16)
    d = d0_v[s16] + d1_v[s16]
    h_v[s16] = h_v[s16] * (1.0 / 20000.0) * _rsqrt16(d)
    return 0
  lax.fori_loop(0, SL // 16, hcomp, 0)
  pltpu.sync_copy(h_v, spm_h.at[sl])

  def zb(i, _):
    d0_v[pl.ds(i * 16, 16)] = jnp.zeros((16,), jnp.float32)
    return 0
  lax.fori_loop(0, SL // 16, zb, 0)
  pltpu.sync_copy(d0_v, spm_agg.at[sl])
  plsc.subcore_barrier()

  def chunk4(i, _):
    for q in range(U):
      ci = i * U + q

      @pl.when(ci < cnt)
      def _():
        pltpu.make_async_copy(edges.at[0, pl.ds(0, CH)], idx_s[q],
                              sem_in).wait()
        pltpu.async_copy(spm_h.at[idx_s[q]], val_v[q], sem_g)
        pltpu.make_async_copy(edges.at[0, pl.ds(0, CH)], idx_d[q],
                              sem_in).wait()
    for q in range(U):
      ci = i * U + q

      @pl.when(ci < cnt)
      def _():
        pltpu.make_async_copy(spm_h.at[idx_s[q]], val_v[q], sem_g).wait()
        pltpu.async_copy(val_v[q], spm_agg.at[idx_d[q]], sem_w, add=True)
    for q in range(U):
      ci = i * U + q

      @pl.when(ci < cnt)
      def _():
        pltpu.make_async_copy(val_v[q], spm_agg.at[idx_d[q]],
                              sem_w).wait()

    @pl.when(i + 1 < NI)
    def _():
      start_loads(i + 1)
    return 0
  lax.fori_loop(0, NI, chunk4, 0)

  plsc.subcore_barrier()
  pltpu.sync_copy(spm_agg.at[sl], agg_hbm.at[pl.ds(cid * NPAD + sid * SL, SL)])


_R4 = 1024
_G4 = NPAD // _R4
_HB = NPAD // 128


def _out_body(a0_ref, a1_ref, di0_ref, di1_ref, w_ref, out_ref):
  a = a0_ref[...] + a1_ref[...]
  d = di0_ref[...] + di1_ref[...]
  av = a * lax.rsqrt(jnp.maximum(d, 1.0))
  w128 = jnp.broadcast_to(w_ref[...], (128, 128))
  rr = lax.broadcasted_iota(jnp.int32, (128, 128), 0)
  cc = lax.broadcasted_iota(jnp.int32, (128, 128), 1)
  eye = rr == cc
  for s in range(_R4 // 128):
    m = jnp.broadcast_to(av[s:s + 1, :], (128, 128))
    dg = jnp.where(eye, m, 0.0)
    blk = lax.dot_general(dg, w128, (((1,), (0,)), ((), ())),
                          preferred_element_type=jnp.float32)
    out_ref[pl.ds(s * 128, 128), :] = blk


def kernel(read_length, edge_index, W):
  degout, degin = _hist_kernel(edge_index)

  rl = jnp.zeros((NPAD,), jnp.float32).at[:N].set(read_length)
  agg = _agg_kernel(edge_index, rl, degout).reshape(2 * _HB, 128)
  degin = degin.reshape(2 * _HB, 128)

  blk8 = _R4 // 128
  out = pl.pallas_call(
      _out_body,
      grid=(_G4,),
      in_specs=[
          pl.BlockSpec((blk8, 128), lambda g: (g, 0)),
          pl.BlockSpec((blk8, 128), lambda g: (g + _HB // blk8, 0)),
          pl.BlockSpec((blk8, 128), lambda g: (g, 0)),
          pl.BlockSpec((blk8, 128), lambda g: (g + _HB // blk8, 0)),
          pl.BlockSpec((1, 128), lambda g: (0, 0)),
      ],
      out_specs=pl.BlockSpec((_R4, 128), lambda g: (g, 0)),
      out_shape=jax.ShapeDtypeStruct((N, D), jnp.float32),
  )(agg, agg, degin, degin, W)
  return out


# speedup vs baseline: 160.0961x; 1.0553x over previous
import functools

import jax
import jax.numpy as jnp
from jax import lax
from jax.experimental import pallas as pl
from jax.experimental.pallas import tpu as pltpu
from jax.experimental.pallas import tpu_sc as plsc

N = 100000
E = 3200000
D = 128

NC = 2
NS = 16
NW = NC * NS

CH = 1024
NCH = E // CH
U = 25
NI = (NCH + NW * U - 1) // (NW * U)

NPAD = 100352
SL = NPAD // NS

assert E % CH == 0 and NPAD % (NS * 8) == 0 and N <= NPAD


def _rsqrt16(d):
  d = jnp.maximum(d, 1.0)
  i = plsc.bitcast(d, jnp.int32)
  y = plsc.bitcast(0x5F3759DF - (i >> 1), jnp.float32)
  for _ in range(3):
    y = y * (1.5 - 0.5 * d * y * y)
  return y


_MESH = plsc.VectorSubcoreMesh(
    core_axis_name="c", subcore_axis_name="s", num_cores=NC, num_subcores=NS)

_SC_PARAMS = pltpu.CompilerParams(
    needs_layout_passes=False, use_tc_tiling_on_sc=False)


@functools.partial(
    pl.kernel,
    out_type=(
        jax.ShapeDtypeStruct((NC * NPAD,), jnp.float32),
        jax.ShapeDtypeStruct((NC * NPAD,), jnp.float32),
    ),
    mesh=_MESH,
    compiler_params=_SC_PARAMS,
    scratch_types=dict(
        idx_s=[pltpu.VMEM((CH,), jnp.int32) for _ in range(U)],
        idx_d=[pltpu.VMEM((CH,), jnp.int32) for _ in range(U)],
        ones_v=pltpu.VMEM((CH,), jnp.float32),
        zbuf=pltpu.VMEM((SL,), jnp.float32),
        spm_out=pltpu.VMEM_SHARED((NPAD,), jnp.float32),
        spm_in=pltpu.VMEM_SHARED((NPAD,), jnp.float32),
        sem_in=pltpu.SemaphoreType.DMA,
        sem_w=pltpu.SemaphoreType.DMA,
    ),
)
def _hist_kernel(edges, degout_hbm, degin_hbm,
                 idx_s, idx_d, ones_v, zbuf, spm_out, spm_in, sem_in, sem_w):
  cid = lax.axis_index("c")
  sid = lax.axis_index("s")
  w = cid * NS + sid
  cnt = (NCH - w + NW - 1) // NW

  def start_loads(j):
    for q in range(U):
      ci = j * U + q
      base = (w + ci * NW) * CH

      @pl.when(ci < cnt)
      def _():
        pltpu.async_copy(edges.at[0, pl.ds(base, CH)], idx_s[q], sem_in)
        pltpu.async_copy(edges.at[1, pl.ds(base, CH)], idx_d[q], sem_in)

  start_loads(0)

  def ob(i, _):
    ones_v[pl.ds(i * 16, 16)] = jnp.ones((16,), jnp.float32)
    return 0
  lax.fori_loop(0, CH // 16, ob, 0)

  def zb(i, _):
    zbuf[pl.ds(i * 16, 16)] = jnp.zeros((16,), jnp.float32)
    return 0
  lax.fori_loop(0, SL // 16, zb, 0)
  sl = pl.ds(sid * SL, SL)
  pltpu.sync_copy(zbuf, spm_out.at[sl])
  pltpu.sync_copy(zbuf, spm_in.at[sl])
  plsc.subcore_barrier()

  def chunk4(i, _):
    for q in range(U):
      ci = i * U + q

      @pl.when(ci < cnt)
      def _():
        pltpu.make_async_copy(edges.at[0, pl.ds(0, CH)], idx_s[q],
                              sem_in).wait()
        pltpu.make_async_copy(edges.at[0, pl.ds(0, CH)], idx_d[q],
                              sem_in).wait()
        pltpu.async_copy(ones_v, spm_out.at[idx_s[q]], sem_w, add=True)
        pltpu.async_copy(ones_v, spm_in.at[idx_d[q]], sem_w, add=True)
    for q in range(U):
      ci = i * U + q

      @pl.when(ci < cnt)
      def _():
        pltpu.make_async_copy(ones_v, spm_out.at[idx_s[q]], sem_w).wait()
        pltpu.make_async_copy(ones_v, spm_in.at[idx_d[q]], sem_w).wait()

    @pl.when(i + 1 < NI)
    def _():
      start_loads(i + 1)
    return 0
  lax.fori_loop(0, NI, chunk4, 0)

  plsc.subcore_barrier()
  osl = pl.ds(cid * NPAD + sid * SL, SL)
  pltpu.sync_copy(spm_out.at[sl], degout_hbm.at[osl])
  pltpu.sync_copy(spm_in.at[sl], degin_hbm.at[osl])


@functools.partial(
    pl.kernel,
    out_type=jax.ShapeDtypeStruct((NC * NPAD,), jnp.float32),
    mesh=_MESH,
    compiler_params=_SC_PARAMS,
    scratch_types=dict(
        idx_s=[pltpu.VMEM((CH,), jnp.int32) for _ in range(U)],
        idx_d=[pltpu.VMEM((CH,), jnp.int32) for _ in range(U)],
        val_v=[pltpu.VMEM((CH,), jnp.float32) for _ in range(U)],
        d0_v=pltpu.VMEM((SL,), jnp.float32),
        d1_v=pltpu.VMEM((SL,), jnp.float32),
        h_v=pltpu.VMEM((SL,), jnp.float32),
        spm_h=pltpu.VMEM_SHARED((NPAD,), jnp.float32),
        spm_agg=pltpu.VMEM_SHARED((NPAD,), jnp.float32),
        sem_in=pltpu.SemaphoreType.DMA,
        sem_g=pltpu.SemaphoreType.DMA,
        sem_w=pltpu.SemaphoreType.DMA,
    ),
)
def _agg_kernel(edges, rl_hbm, degout_hbm, agg_hbm,
                idx_s, idx_d, val_v, d0_v, d1_v, h_v,
                spm_h, spm_agg, sem_in, sem_g, sem_w):
  cid = lax.axis_index("c")
  sid = lax.axis_index("s")
  w = cid * NS + sid
  cnt = (NCH - w + NW - 1) // NW

  def start_loads(j):
    for q in range(U):
      ci = j * U + q
      base = (w + ci * NW) * CH

      @pl.when(ci < cnt)
      def _():
        pltpu.async_copy(edges.at[0, pl.ds(base, CH)], idx_s[q], sem_in)
        pltpu.async_copy(edges.at[1, pl.ds(base, CH)], idx_d[q], sem_in)

  start_loads(0)

  sl = pl.ds(sid * SL, SL)
  pltpu.sync_copy(degout_hbm.at[pl.ds(sid * SL, SL)], d0_v)
  pltpu.sync_copy(degout_hbm.at[pl.ds(NPAD + sid * SL, SL)], d1_v)
  pltpu.sync_copy(rl_hbm.at[sl], h_v)

  def hcomp(i, _):
    s16 = pl.ds(i * 16, 16)
    d = d0_v[s16] + d1_v[s16]
    h_v[s16] = h_v[s16] * (1.0 / 20000.0) * _rsqrt16(d)
    return 0
  lax.fori_loop(0, SL // 16, hcomp, 0)
  pltpu.sync_copy(h_v, spm_h.at[sl])

  def zb(i, _):
    d0_v[pl.ds(i * 16, 16)] = jnp.zeros((16,), jnp.float32)
    return 0
  lax.fori_loop(0, SL // 16, zb, 0)
  pltpu.sync_copy(d0_v, spm_agg.at[sl])
  plsc.subcore_barrier()

  def chunk4(i, _):
    for q in range(U):
      ci = i * U + q

      @pl.when(ci < cnt)
      def _():
        pltpu.make_async_copy(edges.at[0, pl.ds(0, CH)], idx_s[q],
                              sem_in).wait()
        pltpu.async_copy(spm_h.at[idx_s[q]], val_v[q], sem_g)
        pltpu.make_async_copy(edges.at[0, pl.ds(0, CH)], idx_d[q],
                              sem_in).wait()
    for q in range(U):
      ci = i * U + q

      @pl.when(ci < cnt)
      def _():
        pltpu.make_async_copy(spm_h.at[idx_s[q]], val_v[q], sem_g).wait()
        pltpu.async_copy(val_v[q], spm_agg.at[idx_d[q]], sem_w, add=True)
    for q in range(U):
      ci = i * U + q

      @pl.when(ci < cnt)
      def _():
        pltpu.make_async_copy(val_v[q], spm_agg.at[idx_d[q]],
                              sem_w).wait()

    @pl.when(i + 1 < NI)
    def _():
      start_loads(i + 1)
    return 0
  lax.fori_loop(0, NI, chunk4, 0)

  plsc.subcore_barrier()
  pltpu.sync_copy(spm_agg.at[sl], agg_hbm.at[pl.ds(cid * NPAD + sid * SL, SL)])


_R4 = 1024
_G4 = NPAD // _R4
_HB = NPAD // 128


def _out_body(a0_ref, a1_ref, di0_ref, di1_ref, w_ref, out_ref):
  a = a0_ref[...] + a1_ref[...]
  d = di0_ref[...] + di1_ref[...]
  av = a * lax.rsqrt(jnp.maximum(d, 1.0))
  w128 = jnp.broadcast_to(w_ref[...], (128, 128))
  rr = lax.broadcasted_iota(jnp.int32, (128, 128), 0)
  cc = lax.broadcasted_iota(jnp.int32, (128, 128), 1)
  eye = rr == cc
  for s in range(_R4 // 128):
    m = jnp.broadcast_to(av[s:s + 1, :], (128, 128))
    dg = jnp.where(eye, m, 0.0)
    blk = lax.dot_general(dg, w128, (((1,), (0,)), ((), ())),
                          preferred_element_type=jnp.float32)
    out_ref[pl.ds(s * 128, 128), :] = blk


def kernel(read_length, edge_index, W):
  degout, degin = _hist_kernel(edge_index)

  rl = jnp.zeros((NPAD,), jnp.float32).at[:N].set(read_length)
  agg = _agg_kernel(edge_index, rl, degout).reshape(2 * _HB, 128)
  degin = degin.reshape(2 * _HB, 128)

  blk8 = _R4 // 128
  out = pl.pallas_call(
      _out_body,
      grid=(_G4,),
      in_specs=[
          pl.BlockSpec((blk8, 128), lambda g: (g, 0)),
          pl.BlockSpec((blk8, 128), lambda g: (g + _HB // blk8, 0)),
          pl.BlockSpec((blk8, 128), lambda g: (g, 0)),
          pl.BlockSpec((blk8, 128), lambda g: (g + _HB // blk8, 0)),
          pl.BlockSpec((1, 128), lambda g: (0, 0)),
      ],
      out_specs=pl.BlockSpec((_R4, 128), lambda g: (g, 0)),
      out_shape=jax.ShapeDtypeStruct((N, D), jnp.float32),
  )(agg, agg, degin, degin, W)
  return out
